# baseline (speedup 1.0000x reference)
import functools

import jax
import jax.numpy as jnp
from jax import lax
from jax.experimental import pallas as pl
from jax.experimental.pallas import tpu as pltpu, tpu_sc as plsc

TOP_K = 8
BLOCK_M = 1024
N_EXPERTS = 64
LANES = 16


def _probs_kernel(x_ref, w_ref, b_ref, probs_ref, probs_t_ref):
    logits = jnp.dot(x_ref[...], w_ref[...],
                     preferred_element_type=jnp.float32)
    logits = logits + b_ref[...]
    m = jnp.max(logits, axis=-1, keepdims=True)
    e = jnp.exp(logits - m)
    probs = e / jnp.sum(e, axis=-1, keepdims=True)
    probs_ref[...] = probs
    probs_t_ref[...] = probs.T


def _tc_probs(x_flat, W, b2):
    M, C = x_flat.shape
    N = W.shape[-1]
    return pl.pallas_call(
        _probs_kernel,
        grid=(M // BLOCK_M,),
        in_specs=[
            pl.BlockSpec((BLOCK_M, C), lambda i: (i, 0)),
            pl.BlockSpec((C, N), lambda i: (0, 0)),
            pl.BlockSpec((1, N), lambda i: (0, 0)),
        ],
        out_specs=[
            pl.BlockSpec((BLOCK_M, N), lambda i: (i, 0)),
            pl.BlockSpec((N, BLOCK_M), lambda i: (0, i)),
        ],
        out_shape=[
            jax.ShapeDtypeStruct((M, N), jnp.float32),
            jax.ShapeDtypeStruct((N, M), jnp.float32),
        ],
    )(x_flat, W, b2)


def _make_sc_topk(M):
    info = plsc.get_sparse_core_info()
    NC, NS = info.num_cores, info.num_subcores
    NW = NC * NS
    rows_per_w = M // NW
    n_groups = rows_per_w // LANES
    mesh = plsc.VectorSubcoreMesh(core_axis_name="c", subcore_axis_name="s")

    @functools.partial(
        pl.kernel, mesh=mesh,
        out_type=jax.ShapeDtypeStruct((N_EXPERTS, M), jnp.float32),
        scratch_types=[
            pltpu.VMEM((N_EXPERTS, rows_per_w), jnp.float32),
            pltpu.VMEM((N_EXPERTS, rows_per_w), jnp.float32),
        ],
    )
    def sc_topk(pt_hbm, rwt_hbm, slab_v, out_v):
        wid = lax.axis_index("s") * NC + lax.axis_index("c")
        base = wid * rows_per_w
        pltpu.sync_copy(pt_hbm.at[:, pl.ds(base, rows_per_w)], slab_v)

        def body(g, carry):
            off = g * LANES
            cols = [slab_v[e, pl.ds(off, LANES)] for e in range(N_EXPERTS)]
            keys = []
            for e, v in enumerate(cols):
                bits = jnp.bitwise_and(
                    lax.bitcast_convert_type(v, jnp.int32), jnp.int32(~63))
                keys.append(jnp.bitwise_or(bits, jnp.int32(63 - e)))
            work = list(keys)
            sentinel = jnp.int32(-2**31)
            for _ in range(TOP_K):
                t = work
                while len(t) > 1:
                    nxt = [jnp.maximum(t[i], t[i + 1])
                           for i in range(0, len(t) - 1, 2)]
                    if len(t) % 2:
                        nxt.append(t[-1])
                    t = nxt
                mx = t[0]
                work = [jnp.where(w == mx, sentinel, w) for w in work]
            for e in range(N_EXPERTS):
                sel = work[e] != keys[e]
                out_v[e, pl.ds(off, LANES)] = jnp.where(
                    sel, cols[e], jnp.float32(0.0))
            return carry

        lax.fori_loop(0, n_groups, body, 0)
        pltpu.sync_copy(out_v, rwt_hbm.at[:, pl.ds(base, rows_per_w)])

    return sc_topk


@jax.jit
def kernel(x, W, b):
    C = x.shape[-1]
    x_flat = x.reshape(-1, C)
    M = x_flat.shape[0]
    b2 = b.reshape(1, W.shape[-1])
    probs, probs_t = _tc_probs(x_flat, W, b2)
    rw_t = _make_sc_topk(M)(probs_t)
    return rw_t.T, probs
